# Initial kernel scaffold; baseline (speedup 1.0000x reference)
import jax
import jax.numpy as jnp
from jax.experimental import pallas as pl


def kernel(features, G, eval_indices):
    raise NotImplementedError("write your pallas kernel here")


# speedup vs baseline: 3.3489x; 3.3489x over previous
import functools

import jax
import jax.numpy as jnp
from jax import lax
from jax.experimental import pallas as pl
from jax.experimental.pallas import tpu as pltpu
from jax.experimental.pallas import tpu_sc as plsc

N_OUT = 2500
N_OUT_PAD = 2560
IN_BITS = 14
MAXL = 128


def _conflict_free_stream(eval_indices, n_out):
    out_idx = eval_indices[0]
    in_idx = eval_indices[1]
    K = out_idx.shape[0]
    K_PAD = ((K + MAXL * 16 + 15) // 16) * 16

    off = jnp.searchsorted(
        out_idx, jnp.arange(n_out + 1, dtype=jnp.int32)).astype(jnp.int32)
    seg_len = off[1:] - off[:-1]
    p = jnp.arange(K, dtype=jnp.int32) - off[out_idx]
    alive = seg_len[None, :] > jnp.arange(MAXL, dtype=jnp.int32)[:, None]
    n_r = jnp.sum(alive, axis=1).astype(jnp.int32)
    rank2d = jnp.cumsum(alive, axis=1).astype(jnp.int32)
    base = jnp.concatenate([
        jnp.zeros(1, jnp.int32),
        jnp.cumsum(((n_r + 15) // 16) * 16)[:-1].astype(jnp.int32),
    ])
    rank = rank2d.reshape(-1)[p * n_out + out_idx] - 1
    slot = base[p] + rank
    packed = (out_idx << IN_BITS) | in_idx
    sentinel = (N_OUT_PAD - 1) << IN_BITS
    return jnp.full((K_PAD,), sentinel, jnp.int32).at[slot].set(packed)


def _sc_segment_sum(F2, stream):
    R, N = F2.shape
    K_PAD = stream.shape[0]
    CH = K_PAD // 16
    U = next(u for u in range(10, 0, -1) if CH % u == 0)
    NI = CH // U

    info = plsc.get_sparse_core_info()
    NC = info.num_cores
    NW = NC * info.num_subcores
    RW = R // NW
    assert R % NW == 0 and RW % 4 == 0
    NP2 = RW // 4

    mesh = plsc.VectorSubcoreMesh(core_axis_name="c", subcore_axis_name="s")

    @functools.partial(
        pl.kernel,
        mesh=mesh,
        compiler_params=pltpu.CompilerParams(needs_layout_passes=False),
        out_type=jax.ShapeDtypeStruct((R, N_OUT_PAD), jnp.float32),
        scratch_types=[
            pltpu.VMEM((K_PAD,), jnp.int32),
            pltpu.VMEM((N,), jnp.float32),
            pltpu.VMEM((N,), jnp.float32),
            pltpu.VMEM((N,), jnp.float32),
            pltpu.VMEM((N,), jnp.float32),
            pltpu.VMEM((N_OUT_PAD,), jnp.float32),
            pltpu.VMEM((N_OUT_PAD,), jnp.float32),
            pltpu.VMEM((N_OUT_PAD,), jnp.float32),
            pltpu.VMEM((N_OUT_PAD,), jnp.float32),
            pltpu.SemaphoreType.DMA,
            pltpu.SemaphoreType.DMA,
        ],
    )
    def seg_kernel(f_hbm, st_hbm, s_hbm, st_v, fv0, fv1, fv2, fv3,
                   av0, av1, av2, av3, sem_f, sem_s):
        fv = [fv0, fv1, fv2, fv3]
        av = [av0, av1, av2, av3]
        wid = lax.axis_index("s") * NC + lax.axis_index("c")
        base = wid * RW
        pltpu.sync_copy(st_hbm, st_v)

        mask_in = jnp.full((16,), (1 << IN_BITS) - 1, jnp.int32)

        def do_pair(fs, as_, row0):
            def zbody(i, carry):
                av[as_][pl.ds(i * 16, 16)] = jnp.zeros((16,), jnp.float32)
                av[as_ + 1][pl.ds(i * 16, 16)] = jnp.zeros((16,), jnp.float32)
                return carry

            lax.fori_loop(0, N_OUT_PAD // 16, zbody, 0)

            def chunk(c):
                pk = st_v[pl.ds(c * 16, 16)]
                iv = lax.bitwise_and(pk, mask_in)
                ov = lax.shift_right_logical(pk, IN_BITS)
                v0 = plsc.load_gather(fv[fs], [iv])
                plsc.addupdate_scatter(av[as_], [ov], v0)
                v1 = plsc.load_gather(fv[fs + 1], [iv])
                plsc.addupdate_scatter(av[as_ + 1], [ov], v1)

            def cbody(i, carry):
                for j in range(U):
                    chunk(i * U + j)
                return carry

            lax.fori_loop(0, NI, cbody, 0)
            for c in range(NI * U, CH):
                chunk(c)
            pltpu.async_copy(av[as_], s_hbm.at[row0], sem_s)
            pltpu.async_copy(av[as_ + 1], s_hbm.at[row0 + 1], sem_s)

        def wait_f(slot):
            pltpu.make_async_copy(f_hbm.at[0], fv[slot], sem_f).wait()

        def wait_s(slot):
            pltpu.make_async_copy(s_hbm.at[0], av[slot], sem_s).wait()

        for q in range(4):
            pltpu.async_copy(f_hbm.at[base + q], fv[q], sem_f)

        def pbody(p2, carry):
            row0 = base + 4 * p2
            wait_f(0)
            wait_f(1)

            @pl.when(p2 > 0)
            def _():
                wait_s(0)
                wait_s(1)

            do_pair(0, 0, row0)

            @pl.when(p2 < NP2 - 1)
            def _():
                pltpu.async_copy(f_hbm.at[row0 + 4], fv[0], sem_f)
                pltpu.async_copy(f_hbm.at[row0 + 5], fv[1], sem_f)

            wait_f(2)
            wait_f(3)

            @pl.when(p2 > 0)
            def _():
                wait_s(2)
                wait_s(3)

            do_pair(2, 2, row0 + 2)

            @pl.when(p2 < NP2 - 1)
            def _():
                pltpu.async_copy(f_hbm.at[row0 + 6], fv[2], sem_f)
                pltpu.async_copy(f_hbm.at[row0 + 7], fv[3], sem_f)

            return carry

        lax.fori_loop(0, NP2, pbody, 0)
        for q in range(4):
            wait_s(q)

    return seg_kernel(F2, stream)


def _tc_apply_g(S, G):
    R = S.shape[0]
    C = G.shape[1]

    def body(g_ref, s_ref, o_ref):
        o_ref[...] = lax.dot(
            g_ref[...], s_ref[...], preferred_element_type=jnp.float32
        )[:, :N_OUT]

    return pl.pallas_call(
        body,
        grid=(R // C,),
        in_specs=[
            pl.BlockSpec((C, C), lambda b: (0, 0)),
            pl.BlockSpec((C, N_OUT_PAD), lambda b: (b, 0)),
        ],
        out_specs=pl.BlockSpec((C, N_OUT), lambda b: (b, 0)),
        out_shape=jax.ShapeDtypeStruct((R, N_OUT), jnp.float32),
    )(G, S)


def kernel(features, G, eval_indices):
    B, Cin, N = features.shape
    F2 = features.reshape(B * Cin, N)
    stream = _conflict_free_stream(eval_indices, N_OUT)
    S = _sc_segment_sum(F2, stream)
    out2 = _tc_apply_g(S, G)
    return out2.reshape(B, G.shape[0], N_OUT)


# speedup vs baseline: 16.5427x; 4.9398x over previous
import functools

import jax
import jax.numpy as jnp
from jax import lax
from jax.experimental import pallas as pl
from jax.experimental.pallas import tpu as pltpu
from jax.experimental.pallas import tpu_sc as plsc

N_OUT = 2500
N_OUT_PAD = 2560
IN_BITS = 14


def _sc_segment_sum(F2, stream):
    R, N = F2.shape
    K_PAD = stream.shape[0]
    CHB = K_PAD // 16
    assert K_PAD % 16 == 0

    info = plsc.get_sparse_core_info()
    NC = info.num_cores
    NW = NC * info.num_subcores
    RW = R // NW
    assert R % NW == 0 and RW % 4 == 0
    NP2 = RW // 4

    mesh = plsc.VectorSubcoreMesh(core_axis_name="c", subcore_axis_name="s")

    @functools.partial(
        pl.kernel,
        mesh=mesh,
        compiler_params=pltpu.CompilerParams(needs_layout_passes=False),
        out_type=jax.ShapeDtypeStruct((R, N_OUT_PAD), jnp.float32),
        scratch_types=[
            pltpu.VMEM((K_PAD,), jnp.int32),
            pltpu.VMEM((N,), jnp.float32),
            pltpu.VMEM((N,), jnp.float32),
            pltpu.VMEM((N,), jnp.float32),
            pltpu.VMEM((N,), jnp.float32),
            pltpu.VMEM((N_OUT_PAD,), jnp.float32),
            pltpu.VMEM((N_OUT_PAD,), jnp.float32),
            pltpu.VMEM((N_OUT_PAD,), jnp.float32),
            pltpu.VMEM((N_OUT_PAD,), jnp.float32),
            pltpu.SemaphoreType.DMA,
            pltpu.SemaphoreType.DMA,
        ],
    )
    def seg_kernel(f_hbm, st_hbm, s_hbm, st_v, fv0, fv1, fv2, fv3,
                   av0, av1, av2, av3, sem_f, sem_s):
        fv = [fv0, fv1, fv2, fv3]
        av = [av0, av1, av2, av3]
        wid = lax.axis_index("s") * NC + lax.axis_index("c")
        base = wid * RW
        pltpu.sync_copy(st_hbm, st_v)

        mask_in = jnp.full((16,), (1 << IN_BITS) - 1, jnp.int32)
        lane_off = jnp.arange(16, dtype=jnp.int32) * CHB
        zeros16 = jnp.zeros((16,), jnp.float32)

        def do_pair(fs, as_, row0):
            @plsc.parallel_loop(0, N_OUT_PAD // 16, 1, unroll=4)
            def _(i):
                av[as_][pl.ds(i * 16, 16)] = zeros16
                av[as_ + 1][pl.ds(i * 16, 16)] = zeros16

            @plsc.parallel_loop(0, CHB, 1, unroll=8)
            def _(c):
                pk = plsc.load_gather(st_v, [lane_off + c])
                iv = lax.bitwise_and(pk, mask_in)
                ov = lax.shift_right_logical(pk, IN_BITS)
                v0 = plsc.load_gather(fv[fs], [iv])
                plsc.addupdate_scatter(av[as_], [ov], v0)
                v1 = plsc.load_gather(fv[fs + 1], [iv])
                plsc.addupdate_scatter(av[as_ + 1], [ov], v1)

            pltpu.async_copy(av[as_], s_hbm.at[row0], sem_s)
            pltpu.async_copy(av[as_ + 1], s_hbm.at[row0 + 1], sem_s)

        def wait_f(slot):
            pltpu.make_async_copy(f_hbm.at[0], fv[slot], sem_f).wait()

        def wait_s(slot):
            pltpu.make_async_copy(s_hbm.at[0], av[slot], sem_s).wait()

        for q in range(4):
            pltpu.async_copy(f_hbm.at[base + q], fv[q], sem_f)

        def pbody(p2, carry):
            row0 = base + 4 * p2
            wait_f(0)
            wait_f(1)

            @pl.when(p2 > 0)
            def _():
                wait_s(0)
                wait_s(1)

            do_pair(0, 0, row0)

            @pl.when(p2 < NP2 - 1)
            def _():
                pltpu.async_copy(f_hbm.at[row0 + 4], fv[0], sem_f)
                pltpu.async_copy(f_hbm.at[row0 + 5], fv[1], sem_f)

            wait_f(2)
            wait_f(3)

            @pl.when(p2 > 0)
            def _():
                wait_s(2)
                wait_s(3)

            do_pair(2, 2, row0 + 2)

            @pl.when(p2 < NP2 - 1)
            def _():
                pltpu.async_copy(f_hbm.at[row0 + 6], fv[2], sem_f)
                pltpu.async_copy(f_hbm.at[row0 + 7], fv[3], sem_f)

            return carry

        lax.fori_loop(0, NP2, pbody, 0)
        for q in range(4):
            wait_s(q)

    return seg_kernel(F2, stream)


def _tc_apply_g(S, G):
    R = S.shape[0]
    C = G.shape[1]

    def body(g_ref, s_ref, o_ref):
        o_ref[...] = lax.dot(
            g_ref[...], s_ref[...], preferred_element_type=jnp.float32
        )[:, :N_OUT]

    return pl.pallas_call(
        body,
        grid=(R // C,),
        in_specs=[
            pl.BlockSpec((C, C), lambda b: (0, 0)),
            pl.BlockSpec((C, N_OUT_PAD), lambda b: (b, 0)),
        ],
        out_specs=pl.BlockSpec((C, N_OUT), lambda b: (b, 0)),
        out_shape=jax.ShapeDtypeStruct((R, N_OUT), jnp.float32),
    )(G, S)


def kernel(features, G, eval_indices):
    B, Cin, N = features.shape
    F2 = features.reshape(B * Cin, N)
    packed = (eval_indices[0] << IN_BITS) | eval_indices[1]
    K = packed.shape[0]
    CHB = (K + 15) // 16
    CHB += 1 - (CHB % 2)
    K_PAD = CHB * 16
    sentinel = (N_OUT_PAD - 1) << IN_BITS
    stream = jnp.concatenate(
        [packed, jnp.full((K_PAD - K,), sentinel, jnp.int32)])
    S = _sc_segment_sum(F2, stream)
    out2 = _tc_apply_g(S, G)
    return out2.reshape(B, G.shape[0], N_OUT)


# speedup vs baseline: 16.7146x; 1.0104x over previous
import functools

import jax
import jax.numpy as jnp
from jax import lax
from jax.experimental import pallas as pl
from jax.experimental.pallas import tpu as pltpu
from jax.experimental.pallas import tpu_sc as plsc

N_OUT = 2500
N_OUT_PAD = 2560
IN_BITS = 14


def _sc_segment_sum(F2, stream):
    R, N = F2.shape
    K_PAD = stream.shape[0]
    CHB = K_PAD // 16
    assert K_PAD % 16 == 0

    info = plsc.get_sparse_core_info()
    NC = info.num_cores
    NW = NC * info.num_subcores
    RW = R // NW
    assert R % NW == 0 and RW % 4 == 0
    NP2 = RW // 4

    mesh = plsc.VectorSubcoreMesh(core_axis_name="c", subcore_axis_name="s")

    @functools.partial(
        pl.kernel,
        mesh=mesh,
        compiler_params=pltpu.CompilerParams(needs_layout_passes=False),
        out_type=jax.ShapeDtypeStruct((R, N_OUT_PAD), jnp.float32),
        scratch_types=[
            pltpu.VMEM((K_PAD,), jnp.int32),
            pltpu.VMEM((N,), jnp.float32),
            pltpu.VMEM((N,), jnp.float32),
            pltpu.VMEM((N,), jnp.float32),
            pltpu.VMEM((N,), jnp.float32),
            pltpu.VMEM((N_OUT_PAD,), jnp.float32),
            pltpu.VMEM((N_OUT_PAD,), jnp.float32),
            pltpu.VMEM((N_OUT_PAD,), jnp.float32),
            pltpu.VMEM((N_OUT_PAD,), jnp.float32),
            pltpu.SemaphoreType.DMA,
            pltpu.SemaphoreType.DMA,
        ],
    )
    def seg_kernel(f_hbm, st_hbm, s_hbm, st_v, fv0, fv1, fv2, fv3,
                   av0, av1, av2, av3, sem_f, sem_s):
        fv = [fv0, fv1, fv2, fv3]
        av = [av0, av1, av2, av3]
        wid = lax.axis_index("s") * NC + lax.axis_index("c")
        base = wid * RW
        pltpu.sync_copy(st_hbm, st_v)

        mask_in = jnp.full((16,), (1 << IN_BITS) - 1, jnp.int32)
        lane_off = jnp.arange(16, dtype=jnp.int32) * CHB
        zeros16 = jnp.zeros((16,), jnp.float32)

        def do_pair(fs, as_, row0):
            @plsc.parallel_loop(0, N_OUT_PAD // 16, 1, unroll=4)
            def _(i):
                av[as_][pl.ds(i * 16, 16)] = zeros16
                av[as_ + 1][pl.ds(i * 16, 16)] = zeros16

            @plsc.parallel_loop(0, CHB, 1, unroll=12)
            def _(c):
                pk = plsc.load_gather(st_v, [lane_off + c])
                iv = lax.bitwise_and(pk, mask_in)
                ov = lax.shift_right_logical(pk, IN_BITS)
                v0 = plsc.load_gather(fv[fs], [iv])
                plsc.addupdate_scatter(av[as_], [ov], v0)
                v1 = plsc.load_gather(fv[fs + 1], [iv])
                plsc.addupdate_scatter(av[as_ + 1], [ov], v1)

            pltpu.async_copy(av[as_], s_hbm.at[row0], sem_s)
            pltpu.async_copy(av[as_ + 1], s_hbm.at[row0 + 1], sem_s)

        def wait_f(slot):
            pltpu.make_async_copy(f_hbm.at[0], fv[slot], sem_f).wait()

        def wait_s(slot):
            pltpu.make_async_copy(s_hbm.at[0], av[slot], sem_s).wait()

        for q in range(4):
            pltpu.async_copy(f_hbm.at[base + q], fv[q], sem_f)

        def pbody(p2, carry):
            row0 = base + 4 * p2
            wait_f(0)
            wait_f(1)

            @pl.when(p2 > 0)
            def _():
                wait_s(0)
                wait_s(1)

            do_pair(0, 0, row0)

            @pl.when(p2 < NP2 - 1)
            def _():
                pltpu.async_copy(f_hbm.at[row0 + 4], fv[0], sem_f)
                pltpu.async_copy(f_hbm.at[row0 + 5], fv[1], sem_f)

            wait_f(2)
            wait_f(3)

            @pl.when(p2 > 0)
            def _():
                wait_s(2)
                wait_s(3)

            do_pair(2, 2, row0 + 2)

            @pl.when(p2 < NP2 - 1)
            def _():
                pltpu.async_copy(f_hbm.at[row0 + 6], fv[2], sem_f)
                pltpu.async_copy(f_hbm.at[row0 + 7], fv[3], sem_f)

            return carry

        lax.fori_loop(0, NP2, pbody, 0)
        for q in range(4):
            wait_s(q)

    return seg_kernel(F2, stream)


def _tc_apply_g(S, G):
    R = S.shape[0]
    C = G.shape[1]

    def body(g_ref, s_ref, o_ref):
        o_ref[...] = lax.dot(
            g_ref[...], s_ref[...], preferred_element_type=jnp.float32
        )[:, :N_OUT]

    return pl.pallas_call(
        body,
        grid=(R // C,),
        in_specs=[
            pl.BlockSpec((C, C), lambda b: (0, 0)),
            pl.BlockSpec((C, N_OUT_PAD), lambda b: (b, 0)),
        ],
        out_specs=pl.BlockSpec((C, N_OUT), lambda b: (b, 0)),
        out_shape=jax.ShapeDtypeStruct((R, N_OUT), jnp.float32),
    )(G, S)


def kernel(features, G, eval_indices):
    B, Cin, N = features.shape
    F2 = features.reshape(B * Cin, N)
    packed = (eval_indices[0] << IN_BITS) | eval_indices[1]
    K = packed.shape[0]
    CHB = (K + 15) // 16
    CHB += 1 - (CHB % 2)
    K_PAD = CHB * 16
    sentinel = (N_OUT_PAD - 1) << IN_BITS
    stream = jnp.concatenate(
        [packed, jnp.full((K_PAD - K,), sentinel, jnp.int32)])
    S = _sc_segment_sum(F2, stream)
    out2 = _tc_apply_g(S, G)
    return out2.reshape(B, G.shape[0], N_OUT)
